# Initial kernel scaffold; baseline (speedup 1.0000x reference)
import jax
import jax.numpy as jnp
from jax.experimental import pallas as pl


def kernel(hidden_states, norm_weight, gate_weight):
    raise NotImplementedError("write your pallas kernel here")


# speedup vs baseline: 1.1810x; 1.1810x over previous
import functools

import jax
import jax.numpy as jnp
from jax.experimental import pallas as pl

EPS = 1e-05
RMS_EPS = 1e-06
TOP_K = 8
NUM_EXPERTS = 64


def _router_kernel(x_ref, nw_ref, gw_ref, probs_ref, idx_ref, logits_ref):
    x = x_ref[...]
    var = jnp.mean(x * x, axis=-1, keepdims=True)
    xn = x * jax.lax.rsqrt(var + RMS_EPS) * nw_ref[...]
    logits = jax.lax.dot_general(
        xn, gw_ref[...], (((1,), (1,)), ((), ())),
        preferred_element_type=jnp.float32)
    logits_ref[...] = logits
    m = jnp.max(logits, axis=-1, keepdims=True)
    e = jnp.exp(logits - m)
    p = e / jnp.sum(e, axis=-1, keepdims=True)

    iota = jax.lax.broadcasted_iota(jnp.int32, p.shape, 1)
    work = p
    vals = []
    idxs = []
    for _ in range(TOP_K):
        mv = jnp.max(work, axis=-1, keepdims=True)
        cand = jnp.where(work == mv, iota, NUM_EXPERTS)
        mi = jnp.min(cand, axis=-1, keepdims=True)
        vals.append(mv)
        idxs.append(mi)
        work = jnp.where(iota == mi, -jnp.inf, work)
    topv = jnp.concatenate(vals, axis=-1)
    topi = jnp.concatenate(idxs, axis=-1)
    probs_ref[...] = topv / (jnp.sum(topv, axis=-1, keepdims=True) + EPS)
    idx_ref[...] = topi


@functools.partial(jax.jit, static_argnames=())
def kernel(hidden_states, norm_weight, gate_weight):
    B, S, D = hidden_states.shape
    N = B * S
    E = gate_weight.shape[0]
    x = hidden_states.reshape(N, D)
    nw = norm_weight.reshape(1, D)

    TB = 1024
    grid = (N // TB,)

    probs, idx, logits = pl.pallas_call(
        _router_kernel,
        grid=grid,
        in_specs=[
            pl.BlockSpec((TB, D), lambda i: (i, 0)),
            pl.BlockSpec((1, D), lambda i: (0, 0)),
            pl.BlockSpec((E, D), lambda i: (0, 0)),
        ],
        out_specs=[
            pl.BlockSpec((TB, TOP_K), lambda i: (i, 0)),
            pl.BlockSpec((TB, TOP_K), lambda i: (i, 0)),
            pl.BlockSpec((TB, E), lambda i: (i, 0)),
        ],
        out_shape=[
            jax.ShapeDtypeStruct((N, TOP_K), jnp.float32),
            jax.ShapeDtypeStruct((N, TOP_K), jnp.int32),
            jax.ShapeDtypeStruct((N, E), jnp.float32),
        ],
    )(x, nw, gate_weight)
    return (probs, idx, logits)
